# Initial kernel scaffold; baseline (speedup 1.0000x reference)
import jax
import jax.numpy as jnp
from jax.experimental import pallas as pl


def kernel(x, adj, enc_W, ext_w, pair_W, beta, dec_W):
    raise NotImplementedError("write your pallas kernel here")


# speedup vs baseline: 16.0445x; 16.0445x over previous
import functools

import jax
import jax.numpy as jnp
from jax import lax
from jax.experimental import pallas as pl
from jax.experimental.pallas import tpu as pltpu
from jax.experimental.pallas import tpu_sc as plsc

STEP = 1.0

N = 10000
E = 320000
NW = 32
NCORE = 2
NTILE = 16
EPW = E // NW
CHUNK = 80
NCHUNK = EPW // CHUNK
NP = 10240
ROWS_PT = NP // NTILE
ZROWS = 128
DEGW = 128
CW = 16

def _worker_ids():
    cid = lax.axis_index("c")
    sid = lax.axis_index("s")
    return cid, sid, cid * NTILE + sid


@functools.cache
def _sc_kernels():
    mesh = plsc.VectorSubcoreMesh(core_axis_name="c", subcore_axis_name="s",
                                  num_cores=NCORE, num_subcores=NTILE)

    @functools.partial(
        pl.kernel,
        out_type=jax.ShapeDtypeStruct((NCORE, NP, DEGW), jnp.float32),
        mesh=mesh,
        scratch_types=[
            pltpu.VMEM((NCHUNK, CHUNK), jnp.int32),
            pltpu.VMEM((CHUNK, DEGW), jnp.float32),
            pltpu.VMEM_SHARED((NP, DEGW), jnp.float32),
        ],
    )
    def sc_degree(dst_hbm, ones_hbm, z_hbm, out_hbm, dst_v, ones_v, acc):
        cid, sid, wid = _worker_ids()
        pltpu.sync_copy(dst_hbm.at[wid], dst_v)
        pltpu.sync_copy(ones_hbm, ones_v)
        pltpu.sync_copy(z_hbm, acc.at[pl.ds(sid * ROWS_PT, ROWS_PT)])
        plsc.subcore_barrier()

        def body(j, _):
            pltpu.sync_copy(ones_v, acc.at[dst_v.at[j]], add=True)
            return 0

        lax.fori_loop(0, NCHUNK, body, 0)
        plsc.subcore_barrier()
        pltpu.sync_copy(acc.at[pl.ds(sid * ROWS_PT, ROWS_PT)],
                        out_hbm.at[cid, pl.ds(sid * ROWS_PT, ROWS_PT)])

    @functools.partial(
        pl.kernel,
        out_type=jax.ShapeDtypeStruct((NCORE, NP, 128), jnp.float32),
        mesh=mesh,
        scratch_types=[
            pltpu.VMEM((NCHUNK, CHUNK), jnp.int32),
            pltpu.VMEM((NCHUNK, CHUNK), jnp.int32),
            pltpu.VMEM((CHUNK, 128), jnp.float32),
            pltpu.VMEM_SHARED((NP, 128), jnp.float32),
            pltpu.SemaphoreType.DMA,
        ],
    )
    def sc_edge_agg(m_hbm, src_hbm, dst_hbm, z_hbm, out_hbm,
                    src_v, dst_v, rows_v, acc, sem):
        cid, sid, wid = _worker_ids()
        pltpu.sync_copy(src_hbm.at[wid], src_v)
        pltpu.sync_copy(dst_hbm.at[wid], dst_v)
        pltpu.sync_copy(z_hbm, acc.at[pl.ds(sid * ROWS_PT, ROWS_PT)])
        plsc.subcore_barrier()

        def body(j, _):
            pltpu.async_copy(m_hbm.at[src_v.at[j]], rows_v, sem).wait()
            pltpu.sync_copy(rows_v, acc.at[dst_v.at[j]], add=True)
            return 0

        lax.fori_loop(0, NCHUNK, body, 0)
        plsc.subcore_barrier()
        pltpu.sync_copy(acc.at[pl.ds(sid * ROWS_PT, ROWS_PT)],
                        out_hbm.at[cid, pl.ds(sid * ROWS_PT, ROWS_PT)])

    return sc_degree, sc_edge_agg


BLK = 1000


def _dot(a, b):
    return jax.lax.dot_general(
        a, b, (((1,), (1,)), ((), ())),
        precision=lax.Precision.HIGHEST, preferred_element_type=jnp.float32)


def _dinv_from_counts(c0, c1):
    deg = c0[:, 0:1] + c1[:, 0:1] + 1.0
    return lax.rsqrt(jnp.maximum(deg, 1e-12))


def _tc_encode_body(x_ref, encw_ref, pairw_ref, c0_ref, c1_ref,
                    h_ref, mp_ref):
    dinv = _dinv_from_counts(c0_ref[...], c1_ref[...])
    h = _dot(x_ref[...], encw_ref[...])
    w_s = 0.5 * (pairw_ref[...] + pairw_ref[...].T)
    h_ref[...] = h
    mp_ref[...] = dinv * _dot(h, w_s.T)


def _tc_update_core(h, h0, e0, e1, mp, dinv, extw, beta):
    agg = dinv * (e0 + e1 + mp)
    conv = agg - h * extw - beta * h0
    return h + STEP * jnp.maximum(conv, 0.0)


def _tc_update_mm_body(h_ref, h0_ref, e0_ref, e1_ref, mp_ref, c0_ref, c1_ref,
                       extw_ref, beta_ref, pairw_ref, hn_ref, mn_ref):
    dinv = _dinv_from_counts(c0_ref[...], c1_ref[...])
    hn = _tc_update_core(h_ref[...], h0_ref[...], e0_ref[...], e1_ref[...],
                         mp_ref[...], dinv, extw_ref[...], beta_ref[0, 0])
    w_s = 0.5 * (pairw_ref[...] + pairw_ref[...].T)
    hn_ref[...] = hn
    mn_ref[...] = dinv * _dot(hn, w_s.T)


def _tc_update_dec_body(h_ref, h0_ref, e0_ref, e1_ref, mp_ref, c0_ref, c1_ref,
                        extw_ref, beta_ref, decw_ref, out_ref):
    dinv = _dinv_from_counts(c0_ref[...], c1_ref[...])
    hn = _tc_update_core(h_ref[...], h0_ref[...], e0_ref[...], e1_ref[...],
                         mp_ref[...], dinv, extw_ref[...], beta_ref[0, 0])
    logits = _dot(hn, decw_ref[...])
    mx = jnp.max(logits, axis=1, keepdims=True)
    sh = logits - mx
    lse = jnp.log(jnp.sum(jnp.exp(sh), axis=1, keepdims=True))
    out_ref[...] = sh - lse


def _row_spec(width):
    return pl.BlockSpec((BLK, width), lambda i: (i, 0))


def _full_spec(shape):
    return pl.BlockSpec(shape, lambda i: tuple(0 for _ in shape))


def _tc_encode(x, enc_W, pair_W, c0, c1):
    return pl.pallas_call(
        _tc_encode_body,
        grid=(N // BLK,),
        in_specs=[_row_spec(128), _full_spec((128, 128)), _full_spec((128, 128)),
                  _row_spec(CW), _row_spec(CW)],
        out_specs=[_row_spec(128), _row_spec(128)],
        out_shape=[jax.ShapeDtypeStruct((N, 128), jnp.float32),
                   jax.ShapeDtypeStruct((N, 128), jnp.float32)],
    )(x, enc_W, pair_W, c0, c1)


def _tc_update_mm(h, h0, e0, e1, mp, c0, c1, extw, beta, pair_W):
    return pl.pallas_call(
        _tc_update_mm_body,
        grid=(N // BLK,),
        in_specs=[_row_spec(128)] * 5 + [_row_spec(CW)] * 2 +
                 [_full_spec((1, 128)), _full_spec((1, 1)),
                  _full_spec((128, 128))],
        out_specs=[_row_spec(128), _row_spec(128)],
        out_shape=[jax.ShapeDtypeStruct((N, 128), jnp.float32),
                   jax.ShapeDtypeStruct((N, 128), jnp.float32)],
    )(h, h0, e0, e1, mp, c0, c1, extw, beta, pair_W)


def _tc_update_dec(h, h0, e0, e1, mp, c0, c1, extw, beta, dec_W):
    return pl.pallas_call(
        _tc_update_dec_body,
        grid=(N // BLK,),
        in_specs=[_row_spec(128)] * 5 + [_row_spec(CW)] * 2 +
                 [_full_spec((1, 128)), _full_spec((1, 1)),
                  _full_spec((64, 128))],
        out_specs=_row_spec(64),
        out_shape=jax.ShapeDtypeStruct((N, 64), jnp.float32),
    )(h, h0, e0, e1, mp, c0, c1, extw, beta, dec_W)


def kernel(x, adj, enc_W, ext_w, pair_W, beta, dec_W):
    src = adj[0].reshape(NW, NCHUNK, CHUNK)
    dst = adj[1].reshape(NW, NCHUNK, CHUNK)
    ones128 = jnp.ones((CHUNK, DEGW), jnp.float32)
    z128 = jnp.zeros((ROWS_PT, 128), jnp.float32)
    extw2 = ext_w.reshape(1, 128)
    beta2 = beta.reshape(1, 1)

    sc_degree, sc_edge_agg = _sc_kernels()
    counts = sc_degree(dst, ones128, z128)
    c0, c1 = counts[0, :N, :CW], counts[1, :N, :CW]
    h, m1 = _tc_encode(x, enc_W, pair_W, c0, c1)
    e = sc_edge_agg(m1, src, dst, z128)
    h1, m2 = _tc_update_mm(h, h, e[0, :N], e[1, :N], m1, c0, c1, extw2, beta2, pair_W)
    e2 = sc_edge_agg(m2, src, dst, z128)
    return _tc_update_dec(h1, h, e2[0, :N], e2[1, :N], m2, c0, c1, extw2, beta2, dec_W)
